# Initial kernel scaffold; baseline (speedup 1.0000x reference)
import jax
import jax.numpy as jnp
from jax.experimental import pallas as pl


def kernel(x, node_attr, target_attr, in_degree, out_degree, node_type_table, node_attr_table, target_table):
    raise NotImplementedError("write your pallas kernel here")


# speedup vs baseline: 15.2799x; 15.2799x over previous
import functools

import jax
import jax.numpy as jnp
from jax import lax
from jax.experimental import pallas as pl
from jax.experimental.pallas import tpu as pltpu
from jax.experimental.pallas import tpu_sc as plsc

N_GRAPH = 1024
N_NODE = 200
HIDDEN = 128

NC = 2
NS = 16
NW = NC * NS

B = N_GRAPH * N_NODE
GRP = 64
N_GROUPS = B // GRP
GROUPS_PER_W = N_GROUPS // NW
K = 10
LEAD = 2


def _feature_kernel(x_idx, a_idx, t_idx, type_tab, attr_tab, tgt_tab, out,
                    xv, av, tv, bufs, tgt_sp, semt, sema, semg, semo):
    wid = lax.axis_index("s") * NC + lax.axis_index("c")
    g0 = wid * GROUPS_PER_W
    @pl.when(lax.axis_index("s") == 0)
    def _():
        pltpu.sync_copy(tgt_tab, tgt_sp)

    pltpu.sync_copy(x_idx.at[wid], xv)
    pltpu.sync_copy(a_idx.at[wid], av)
    pltpu.sync_copy(t_idx.at[wid], tv)
    plsc.subcore_barrier()

    G = GROUPS_PER_W

    @pl.loop(0, G + 3 * LEAD)
    def _(i):
        @pl.when(i >= 3 * LEAD)
        def _():
            g = i - 3 * LEAD
            pltpu.make_async_copy(type_tab.at[pl.ds(0, GRP)], bufs.at[0],
                                  semg).wait()
            pltpu.async_copy(bufs.at[lax.rem(g, K)],
                             out.at[pl.ds((g0 + g) * GRP, GRP)], semo)

        @pl.when((i >= 2 * LEAD) & (i < G + 2 * LEAD))
        def _():
            g = i - 2 * LEAD
            pltpu.make_async_copy(type_tab.at[pl.ds(0, GRP)], bufs.at[0],
                                  sema).wait()
            pltpu.async_copy(tgt_sp.at[tv.at[g]], bufs.at[lax.rem(g, K)],
                             semg, add=True)

        @pl.when((i >= LEAD) & (i < G + LEAD))
        def _():
            g = i - LEAD
            pltpu.make_async_copy(type_tab.at[pl.ds(0, GRP)], bufs.at[0],
                                  semt).wait()
            pltpu.async_copy(attr_tab.at[av.at[g]], bufs.at[lax.rem(g, K)],
                             sema, add=True)

        @pl.when(i < G)
        def _():
            @pl.when(i >= K)
            def _():
                pltpu.make_async_copy(bufs.at[0], out.at[pl.ds(0, GRP)],
                                      semo).wait()
            pltpu.async_copy(type_tab.at[xv.at[i]], bufs.at[lax.rem(i, K)],
                             semt)

    for _i in range(K):
        pltpu.make_async_copy(bufs.at[0], out.at[pl.ds(0, GRP)], semo).wait()


@jax.jit
def _run(x_idx, a_idx, t_idx, type_tab, attr_tab, tgt_tab):
    mesh = plsc.VectorSubcoreMesh(core_axis_name="c", subcore_axis_name="s")
    return pl.kernel(
        _feature_kernel,
        out_type=jax.ShapeDtypeStruct((B, HIDDEN), jnp.float32),
        mesh=mesh,
        scratch_types=[
            pltpu.VMEM((GROUPS_PER_W, GRP), jnp.int32),
            pltpu.VMEM((GROUPS_PER_W, GRP), jnp.int32),
            pltpu.VMEM((GROUPS_PER_W, GRP), jnp.int32),
            pltpu.VMEM((K, GRP, HIDDEN), jnp.float32),
            pltpu.VMEM_SHARED((2 * TGT_REP, HIDDEN), jnp.float32),
            pltpu.SemaphoreType.DMA,
            pltpu.SemaphoreType.DMA,
            pltpu.SemaphoreType.DMA,
            pltpu.SemaphoreType.DMA,
        ],
    )(x_idx, a_idx, t_idx, type_tab, attr_tab, tgt_tab)


TGT_REP = 256


def kernel(x, node_attr, target_attr, in_degree, out_degree,
           node_type_table, node_attr_table, target_table):
    del in_degree, out_degree
    x_idx = x.reshape(NW, GROUPS_PER_W, GRP).astype(jnp.int32)
    a_idx = node_attr.reshape(NW, GROUPS_PER_W, GRP).astype(jnp.int32)
    tgt_big = jnp.tile(target_table, (TGT_REP, 1))
    spread = (jnp.arange(B, dtype=jnp.int32) % TGT_REP) * 2
    t_idx = (target_attr.reshape(B).astype(jnp.int32) + spread
             ).reshape(NW, GROUPS_PER_W, GRP)
    out = _run(x_idx, a_idx, t_idx, node_type_table, node_attr_table,
               tgt_big)
    return out.reshape(N_GRAPH, N_NODE, HIDDEN)


# speedup vs baseline: 15.5248x; 1.0160x over previous
import functools

import jax
import jax.numpy as jnp
from jax import lax
from jax.experimental import pallas as pl
from jax.experimental.pallas import tpu as pltpu
from jax.experimental.pallas import tpu_sc as plsc

N_GRAPH = 1024
N_NODE = 200
HIDDEN = 128

NC = 2
NS = 16
NW = NC * NS

B = N_GRAPH * N_NODE
GRP = 64
N_GROUPS = B // GRP
GROUPS_PER_W = N_GROUPS // NW
K = 10
LEAD = 3


def _feature_kernel(x_idx, a_idx, t_idx, type_tab, attr_tab, tgt_tab, out,
                    xv, av, tv, bufs, tgt_sp, semt, sema, semg, semo):
    wid = lax.axis_index("s") * NC + lax.axis_index("c")
    g0 = wid * GROUPS_PER_W
    @pl.when(lax.axis_index("s") == 0)
    def _():
        pltpu.sync_copy(tgt_tab, tgt_sp)

    pltpu.sync_copy(x_idx.at[wid], xv)
    pltpu.sync_copy(a_idx.at[wid], av)
    pltpu.sync_copy(t_idx.at[wid], tv)
    plsc.subcore_barrier()

    G = GROUPS_PER_W

    @pl.loop(0, G + 3 * LEAD)
    def _(i):
        @pl.when(i >= 3 * LEAD)
        def _():
            g = i - 3 * LEAD
            pltpu.make_async_copy(type_tab.at[pl.ds(0, GRP)], bufs.at[0],
                                  semg).wait()
            pltpu.async_copy(bufs.at[lax.rem(g, K)],
                             out.at[pl.ds((g0 + g) * GRP, GRP)], semo)

        @pl.when((i >= 2 * LEAD) & (i < G + 2 * LEAD))
        def _():
            g = i - 2 * LEAD
            pltpu.make_async_copy(type_tab.at[pl.ds(0, GRP)], bufs.at[0],
                                  sema).wait()
            pltpu.async_copy(tgt_sp.at[tv.at[g]], bufs.at[lax.rem(g, K)],
                             semg, add=True)

        @pl.when((i >= LEAD) & (i < G + LEAD))
        def _():
            g = i - LEAD
            pltpu.make_async_copy(type_tab.at[pl.ds(0, GRP)], bufs.at[0],
                                  semt).wait()
            pltpu.async_copy(attr_tab.at[av.at[g]], bufs.at[lax.rem(g, K)],
                             sema, add=True)

        @pl.when(i < G)
        def _():
            @pl.when(i >= K)
            def _():
                pltpu.make_async_copy(bufs.at[0], out.at[pl.ds(0, GRP)],
                                      semo).wait()
            pltpu.async_copy(type_tab.at[xv.at[i]], bufs.at[lax.rem(i, K)],
                             semt)

    for _i in range(K):
        pltpu.make_async_copy(bufs.at[0], out.at[pl.ds(0, GRP)], semo).wait()


@jax.jit
def _run(x_idx, a_idx, t_idx, type_tab, attr_tab, tgt_tab):
    mesh = plsc.VectorSubcoreMesh(core_axis_name="c", subcore_axis_name="s")
    return pl.kernel(
        _feature_kernel,
        out_type=jax.ShapeDtypeStruct((B, HIDDEN), jnp.float32),
        mesh=mesh,
        scratch_types=[
            pltpu.VMEM((GROUPS_PER_W, GRP), jnp.int32),
            pltpu.VMEM((GROUPS_PER_W, GRP), jnp.int32),
            pltpu.VMEM((GROUPS_PER_W, GRP), jnp.int32),
            pltpu.VMEM((K, GRP, HIDDEN), jnp.float32),
            pltpu.VMEM_SHARED((2 * TGT_REP, HIDDEN), jnp.float32),
            pltpu.SemaphoreType.DMA,
            pltpu.SemaphoreType.DMA,
            pltpu.SemaphoreType.DMA,
            pltpu.SemaphoreType.DMA,
        ],
    )(x_idx, a_idx, t_idx, type_tab, attr_tab, tgt_tab)


TGT_REP = 256


def kernel(x, node_attr, target_attr, in_degree, out_degree,
           node_type_table, node_attr_table, target_table):
    del in_degree, out_degree
    x_idx = x.reshape(NW, GROUPS_PER_W, GRP).astype(jnp.int32)
    a_idx = node_attr.reshape(NW, GROUPS_PER_W, GRP).astype(jnp.int32)
    tgt_big = jnp.tile(target_table, (TGT_REP, 1))
    spread = (jnp.arange(B, dtype=jnp.int32) % TGT_REP) * 2
    t_idx = (target_attr.reshape(B).astype(jnp.int32) + spread
             ).reshape(NW, GROUPS_PER_W, GRP)
    out = _run(x_idx, a_idx, t_idx, node_type_table, node_attr_table,
               tgt_big)
    return out.reshape(N_GRAPH, N_NODE, HIDDEN)


# speedup vs baseline: 15.9294x; 1.0261x over previous
import functools

import jax
import jax.numpy as jnp
from jax import lax
from jax.experimental import pallas as pl
from jax.experimental.pallas import tpu as pltpu
from jax.experimental.pallas import tpu_sc as plsc

N_GRAPH = 1024
N_NODE = 200
HIDDEN = 128

NC = 2
NS = 16
NW = NC * NS

B = N_GRAPH * N_NODE
GRP = 64
N_GROUPS = B // GRP
GROUPS_PER_W = N_GROUPS // NW
K = 10
LEAD = 3


def _feature_kernel(x_idx, a_idx, t_idx, type_tab, attr_tab, tgt_tab, out,
                    xv, av, tv, bufs, tgt_sp, semt, sema, semg, semo):
    wid = lax.axis_index("s") * NC + lax.axis_index("c")
    sid = lax.axis_index("s")
    g0 = wid * GROUPS_PER_W
    @pl.when(sid == 0)
    def _():
        pltpu.async_copy(tgt_tab, tgt_sp, semg)

    ix = pltpu.async_copy(x_idx.at[wid], xv, semt)
    ia = pltpu.async_copy(a_idx.at[wid], av, semt)
    it = pltpu.async_copy(t_idx.at[wid], tv, semt)
    ix.wait(); ia.wait(); it.wait()

    G = GROUPS_PER_W

    @pl.loop(0, G + 3 * LEAD)
    def _(i):
        @pl.when(i >= 3 * LEAD)
        def _():
            g = i - 3 * LEAD
            pltpu.make_async_copy(type_tab.at[pl.ds(0, GRP)], bufs.at[0],
                                  semg).wait()
            pltpu.async_copy(bufs.at[lax.rem(g, K)],
                             out.at[pl.ds((g0 + g) * GRP, GRP)], semo)

        @pl.when(i == 2 * LEAD)
        def _():
            @pl.when(sid == 0)
            def _():
                pltpu.make_async_copy(tgt_tab, tgt_sp, semg).wait()
            plsc.subcore_barrier()

        @pl.when((i >= 2 * LEAD) & (i < G + 2 * LEAD))
        def _():
            g = i - 2 * LEAD
            pltpu.make_async_copy(type_tab.at[pl.ds(0, GRP)], bufs.at[0],
                                  sema).wait()
            pltpu.async_copy(tgt_sp.at[tv.at[g]], bufs.at[lax.rem(g, K)],
                             semg, add=True)

        @pl.when((i >= LEAD) & (i < G + LEAD))
        def _():
            g = i - LEAD
            pltpu.make_async_copy(type_tab.at[pl.ds(0, GRP)], bufs.at[0],
                                  semt).wait()
            pltpu.async_copy(attr_tab.at[av.at[g]], bufs.at[lax.rem(g, K)],
                             sema, add=True)

        @pl.when(i < G)
        def _():
            @pl.when(i >= K)
            def _():
                pltpu.make_async_copy(bufs.at[0], out.at[pl.ds(0, GRP)],
                                      semo).wait()
            pltpu.async_copy(type_tab.at[xv.at[i]], bufs.at[lax.rem(i, K)],
                             semt)

    for _i in range(K):
        pltpu.make_async_copy(bufs.at[0], out.at[pl.ds(0, GRP)], semo).wait()


@jax.jit
def _run(x_idx, a_idx, t_idx, type_tab, attr_tab, tgt_tab):
    mesh = plsc.VectorSubcoreMesh(core_axis_name="c", subcore_axis_name="s")
    return pl.kernel(
        _feature_kernel,
        out_type=jax.ShapeDtypeStruct((B, HIDDEN), jnp.float32),
        mesh=mesh,
        scratch_types=[
            pltpu.VMEM((GROUPS_PER_W, GRP), jnp.int32),
            pltpu.VMEM((GROUPS_PER_W, GRP), jnp.int32),
            pltpu.VMEM((GROUPS_PER_W, GRP), jnp.int32),
            pltpu.VMEM((K, GRP, HIDDEN), jnp.float32),
            pltpu.VMEM_SHARED((2 * TGT_REP, HIDDEN), jnp.float32),
            pltpu.SemaphoreType.DMA,
            pltpu.SemaphoreType.DMA,
            pltpu.SemaphoreType.DMA,
            pltpu.SemaphoreType.DMA,
        ],
    )(x_idx, a_idx, t_idx, type_tab, attr_tab, tgt_tab)


TGT_REP = 256


def kernel(x, node_attr, target_attr, in_degree, out_degree,
           node_type_table, node_attr_table, target_table):
    del in_degree, out_degree
    x_idx = x.reshape(NW, GROUPS_PER_W, GRP).astype(jnp.int32)
    a_idx = node_attr.reshape(NW, GROUPS_PER_W, GRP).astype(jnp.int32)
    tgt_big = jnp.tile(target_table, (TGT_REP, 1))
    spread = (jnp.arange(B, dtype=jnp.int32) % TGT_REP) * 2
    t_idx = (target_attr.reshape(B).astype(jnp.int32) + spread
             ).reshape(NW, GROUPS_PER_W, GRP)
    out = _run(x_idx, a_idx, t_idx, node_type_table, node_attr_table,
               tgt_big)
    return out.reshape(N_GRAPH, N_NODE, HIDDEN)
